# baseline (device time: 31551 ns/iter reference)
import jax
import jax.numpy as jnp
from jax import lax
from jax.experimental import pallas as pl
from jax.experimental.pallas import tpu as pltpu

N_DEV = 16
BLK = 64
DH = 64


def kernel(x, Wq, K_ext, V_ext, Wo):
    B, Sq, D = x.shape
    Skv = K_ext.shape[1]
    HD = Wq.shape[1]
    H_per = HD // DH
    M = B * Sq
    CHUNK = M // N_DEV
    BH = B * H_per

    def body(x_ref, wq_ref, k_hbm, v_hbm, wo_ref, out_ref,
             acc_ref, comm_ref, stage_ref, k_vmem, v_vmem,
             kv_sems, rs_send, rs_recv, ag_send, ag_recv):
        my = lax.axis_index("i")
        head0 = my * H_per

        kcp = pltpu.make_async_copy(
            k_hbm.at[:, :, pl.ds(head0, H_per), :], k_vmem, kv_sems.at[0])
        vcp = pltpu.make_async_copy(
            v_hbm.at[:, :, pl.ds(head0, H_per), :], v_vmem, kv_sems.at[1])
        kcp.start()
        vcp.start()

        barrier = pltpu.get_barrier_semaphore()
        for j in range(1, N_DEV):
            p = lax.rem(my + j, N_DEV)
            pl.semaphore_signal(barrier, inc=1, device_id=(p,),
                                device_id_type=pl.DeviceIdType.MESH)

        x2 = x_ref[...].reshape(M, D)
        q = jnp.dot(x2, wq_ref[...], preferred_element_type=jnp.float32)
        qb = jnp.transpose(q.reshape(B, Sq, H_per, DH),
                           (0, 2, 1, 3)).reshape(BH, Sq, DH)

        kcp.wait()
        vcp.wait()
        kb = jnp.transpose(k_vmem[...], (0, 2, 1, 3)).reshape(BH, Skv, DH)
        vb = jnp.transpose(v_vmem[...], (0, 2, 1, 3)).reshape(BH, Skv, DH)

        rb = lax.broadcasted_iota(jnp.int32, (Sq, Skv), 0) // BLK
        cb = lax.broadcasted_iota(jnp.int32, (Sq, Skv), 1) // BLK
        mask = (cb <= rb)[None, :, :]

        s = lax.dot_general(
            qb, kb, (((2,), (2,)), ((0,), (0,))),
            preferred_element_type=jnp.float32) * 0.125
        w = jnp.exp(jnp.where(mask, s, -1e9))
        denom = jnp.sum(w, axis=2, keepdims=True)
        ctx = lax.dot_general(
            w, vb, (((2,), (1,)), ((0,), (0,))),
            preferred_element_type=jnp.float32) / denom
        ctx2 = jnp.transpose(ctx.reshape(B, H_per, Sq, DH),
                             (0, 2, 1, 3)).reshape(M, HD)

        acc_ref[...] = jnp.dot(ctx2, wo_ref[...],
                               preferred_element_type=jnp.float32)
        comm_ref[...] = acc_ref[...].astype(jnp.bfloat16)

        pl.semaphore_wait(barrier, N_DEV - 1)
        pending_sends = []

        r_rdmas = []
        for j in range(1, N_DEV):
            p = lax.rem(my + j, N_DEV)
            rdma = pltpu.make_async_remote_copy(
                src_ref=comm_ref.at[pl.ds(p * CHUNK, CHUNK)],
                dst_ref=stage_ref.at[j - 1],
                send_sem=rs_send.at[j - 1],
                recv_sem=rs_recv.at[j - 1],
                device_id=(p,),
                device_id_type=pl.DeviceIdType.MESH,
            )
            rdma.start()
            r_rdmas.append(rdma)
        for rdma in r_rdmas:
            rdma.wait_recv()
        pending_sends += r_rdmas

        my_lo = my * CHUNK
        reduced = (acc_ref[pl.ds(my_lo, CHUNK), :]
                   + jnp.sum(stage_ref[...].astype(jnp.float32), axis=0))
        comm_ref[pl.ds(my_lo, CHUNK), :] = reduced.astype(jnp.bfloat16)

        b_rdmas = []
        for j in range(1, N_DEV):
            p = lax.rem(my + j, N_DEV)
            rdma = pltpu.make_async_remote_copy(
                src_ref=comm_ref.at[pl.ds(my_lo, CHUNK)],
                dst_ref=comm_ref.at[pl.ds(my_lo, CHUNK)],
                send_sem=ag_send.at[j - 1],
                recv_sem=ag_recv.at[j - 1],
                device_id=(p,),
                device_id_type=pl.DeviceIdType.MESH,
            )
            rdma.start()
            b_rdmas.append(rdma)
        for rdma in b_rdmas:
            rdma.wait_recv()
        pending_sends += b_rdmas

        for rdma in pending_sends:
            rdma.wait_send()

        out_ref[...] = comm_ref[...].astype(jnp.float32).reshape(B, Sq, D)

    return pl.pallas_call(
        body,
        out_shape=jax.ShapeDtypeStruct((B, Sq, D), jnp.float32),
        in_specs=[
            pl.BlockSpec(memory_space=pltpu.VMEM),
            pl.BlockSpec(memory_space=pltpu.VMEM),
            pl.BlockSpec(memory_space=pltpu.HBM),
            pl.BlockSpec(memory_space=pltpu.HBM),
            pl.BlockSpec(memory_space=pltpu.VMEM),
        ],
        out_specs=pl.BlockSpec(memory_space=pltpu.VMEM),
        scratch_shapes=[
            pltpu.VMEM((M, D), jnp.float32),
            pltpu.VMEM((M, D), jnp.bfloat16),
            pltpu.VMEM((N_DEV - 1, CHUNK, D), jnp.bfloat16),
            pltpu.VMEM((B, Skv, H_per, DH), jnp.float32),
            pltpu.VMEM((B, Skv, H_per, DH), jnp.float32),
            pltpu.SemaphoreType.DMA((2,)),
            pltpu.SemaphoreType.DMA((N_DEV - 1,)),
            pltpu.SemaphoreType.DMA((N_DEV - 1,)),
            pltpu.SemaphoreType.DMA((N_DEV - 1,)),
            pltpu.SemaphoreType.DMA((N_DEV - 1,)),
        ],
        compiler_params=pltpu.CompilerParams(collective_id=0),
    )(x, Wq, K_ext, V_ext, Wo)


# device time: 19782 ns/iter; 1.5949x vs baseline; 1.5949x over previous
import jax
import jax.numpy as jnp
from jax import lax
from jax.experimental import pallas as pl
from jax.experimental.pallas import tpu as pltpu

N_DEV = 16
BLK = 64
DH = 64


def kernel(x, Wq, K_ext, V_ext, Wo):
    B, Sq, D = x.shape
    Skv = K_ext.shape[1]
    HD = Wq.shape[1]
    H_per = HD // DH
    M = B * Sq
    CHUNK = M // N_DEV

    pos = lax.axis_index("i")
    k_loc = lax.dynamic_slice_in_dim(K_ext, pos * H_per, H_per, axis=2)
    v_loc = lax.dynamic_slice_in_dim(V_ext, pos * H_per, H_per, axis=2)
    k_loc = jnp.moveaxis(k_loc, 2, 1)
    v_loc = jnp.moveaxis(v_loc, 2, 1)

    def body(x_ref, wq_ref, k_ref, v_ref, wo_ref, out_ref,
             acc_ref, ctx_ref, comm_ref, stage_ref,
             rs_send, rs_recv, ag_send, ag_recv):
        my = lax.axis_index("i")

        barrier = pltpu.get_barrier_semaphore()
        for j in range(1, N_DEV):
            p = lax.rem(my + j, N_DEV)
            pl.semaphore_signal(barrier, inc=1, device_id=(p,),
                                device_id_type=pl.DeviceIdType.MESH)

        x2 = x_ref[...].reshape(M, D)
        q = jnp.dot(x2, wq_ref[...], preferred_element_type=jnp.float32)

        rb = lax.broadcasted_iota(jnp.int32, (Sq, Skv), 0) // BLK
        cb = lax.broadcasted_iota(jnp.int32, (Sq, Skv), 1) // BLK
        neg = jnp.where(cb <= rb, 0.0, -1e9)

        r_rdmas = []

        def compute_batch(b):
            for h in range(H_per):
                qh = q[b * Sq:(b + 1) * Sq, h * DH:(h + 1) * DH]
                kh = k_ref[b, h]
                vh = v_ref[b, h]
                s = lax.dot_general(
                    qh, kh, (((1,), (1,)), ((), ())),
                    preferred_element_type=jnp.float32,
                ) * 0.125 + neg
                w = jnp.exp(s)
                denom = jnp.sum(w, axis=1, keepdims=True)
                ctx_ref[b * Sq:(b + 1) * Sq, h * DH:(h + 1) * DH] = (
                    jnp.dot(w, vh, preferred_element_type=jnp.float32)
                    / denom)
            rows = pl.ds(b * Sq, Sq)
            acc_ref[rows, :] = jnp.dot(
                ctx_ref[rows, :], wo_ref[...],
                preferred_element_type=jnp.float32)
            comm_ref[rows, :] = acc_ref[rows, :].astype(jnp.bfloat16)

        def send_batch(b, peers_lo, peers_hi):
            for j in range(1, N_DEV):
                p = lax.rem(my + j, N_DEV)
                rdma = pltpu.make_async_remote_copy(
                    src_ref=comm_ref.at[pl.ds(p * CHUNK, CHUNK)],
                    dst_ref=stage_ref.at[j - 1],
                    send_sem=rs_send.at[j - 1],
                    recv_sem=rs_recv.at[j - 1],
                    device_id=(p,),
                    device_id_type=pl.DeviceIdType.MESH,
                )

                @pl.when(jnp.logical_and(p >= peers_lo, p < peers_hi))
                def _():
                    rdma.start()

                if b == B - 1:
                    r_rdmas.append(rdma)

        peers_per_batch = Sq // CHUNK
        compute_batch(0)
        pl.semaphore_wait(barrier, N_DEV - 1)
        send_batch(0, 0, peers_per_batch)
        compute_batch(1)
        send_batch(1, peers_per_batch, N_DEV)

        pending_sends = r_rdmas[:]
        my_lo = my * CHUNK
        reduced = acc_ref[pl.ds(my_lo, CHUNK), :]
        wait_order = [1, 3, 4, 12, 13, 15, 5, 8, 11, 2, 14, 7, 9, 6, 10]
        for j in wait_order:
            r_rdmas[j - 1].wait_recv()
            reduced = reduced + stage_ref[j - 1].astype(jnp.float32)
        comm_ref[pl.ds(my_lo, CHUNK), :] = reduced.astype(jnp.bfloat16)

        b_rdmas = []
        for j in range(1, N_DEV):
            p = lax.rem(my + j, N_DEV)
            rdma = pltpu.make_async_remote_copy(
                src_ref=comm_ref.at[pl.ds(my_lo, CHUNK)],
                dst_ref=comm_ref.at[pl.ds(my_lo, CHUNK)],
                send_sem=ag_send.at[j - 1],
                recv_sem=ag_recv.at[j - 1],
                device_id=(p,),
                device_id_type=pl.DeviceIdType.MESH,
            )
            rdma.start()
            b_rdmas.append(rdma)
        for rdma in b_rdmas:
            rdma.wait_recv()
        pending_sends += b_rdmas

        out_ref[...] = comm_ref[...].astype(jnp.float32).reshape(B, Sq, D)

        for rdma in pending_sends:
            rdma.wait_send()

    return pl.pallas_call(
        body,
        out_shape=jax.ShapeDtypeStruct((B, Sq, D), jnp.float32),
        in_specs=[pl.BlockSpec(memory_space=pltpu.VMEM)] * 5,
        out_specs=pl.BlockSpec(memory_space=pltpu.VMEM),
        scratch_shapes=[
            pltpu.VMEM((M, D), jnp.float32),
            pltpu.VMEM((M, HD), jnp.float32),
            pltpu.VMEM((M, D), jnp.bfloat16),
            pltpu.VMEM((N_DEV - 1, CHUNK, D), jnp.bfloat16),
            pltpu.SemaphoreType.DMA((N_DEV - 1,)),
            pltpu.SemaphoreType.DMA((N_DEV - 1,)),
            pltpu.SemaphoreType.DMA((N_DEV - 1,)),
            pltpu.SemaphoreType.DMA((N_DEV - 1,)),
        ],
        compiler_params=pltpu.CompilerParams(collective_id=0),
    )(x, Wq, k_loc, v_loc, Wo)
